# Initial kernel scaffold; baseline (speedup 1.0000x reference)
import jax
import jax.numpy as jnp
from jax.experimental import pallas as pl


def kernel(x, W1, g1, b1, W2, g2, b2, W3, g3, b3, Wc, bc, gn, bnb):
    raise NotImplementedError("write your pallas kernel here")


# speedup vs baseline: 29.3317x; 29.3317x over previous
import functools

import jax
import jax.numpy as jnp
from jax import lax
from jax.experimental import pallas as pl
from jax.experimental.pallas import tpu as pltpu
from jax.experimental.pallas import tpu_sc as plsc

KNN = 5
_INTERPRET = False


def _knn_project(ht, WbT, rb=512):
    B, N, C = ht.shape
    O = WbT.shape[1]
    nb = N // rb

    def body(hrow_ref, hfull_ref, wb_ref, idx_ref, v2_ref):
        b = pl.program_id(0)
        r = hrow_ref[0]
        f = hfull_ref[0]
        g = lax.dot_general(r, f, (((1,), (1,)), ((), ())),
                            preferred_element_type=jnp.float32)
        xx_r = jnp.sum(r * r, axis=1, keepdims=True)
        xx_f = jnp.sum(f * f, axis=1).reshape(1, N)
        d = 2.0 * g - xx_r - xx_f
        iota = lax.broadcasted_iota(jnp.int32, (rb, N), 1)
        off = b * N
        for t in range(KNN):
            vmax = jnp.max(d, axis=1, keepdims=True)
            amin = jnp.min(jnp.where(d == vmax, iota, N), axis=1)
            idx_ref[0, t, :] = amin + off
            d = jnp.where(iota == amin[:, None], -jnp.inf, d)
        v2_ref[0] = jnp.dot(r, wb_ref[...], preferred_element_type=jnp.float32)

    return pl.pallas_call(
        body,
        grid=(B, nb),
        in_specs=[
            pl.BlockSpec((1, rb, C), lambda b, i: (b, i, 0)),
            pl.BlockSpec((1, N, C), lambda b, i: (b, 0, 0)),
            pl.BlockSpec((C, O), lambda b, i: (0, 0)),
        ],
        out_specs=[
            pl.BlockSpec((1, KNN, rb), lambda b, i: (b, 0, i)),
            pl.BlockSpec((1, rb, O), lambda b, i: (b, i, 0)),
        ],
        out_shape=[
            jax.ShapeDtypeStruct((B, KNN, N), jnp.int32),
            jax.ShapeDtypeStruct((B, N, O), jnp.float32),
        ],
        interpret=_INTERPRET,
    )(ht, ht, WbT)


def _gather_nbr(hf, idxf):
    M, C = hf.shape
    NW = 32
    PW = M // NW
    P = 64
    SUB = PW // P
    N = M // 2

    mesh = plsc.VectorSubcoreMesh(core_axis_name="c", subcore_axis_name="s")

    @functools.partial(
        pl.kernel,
        mesh=mesh,
        out_type=jax.ShapeDtypeStruct((KNN, M, C), jnp.float32),
        scratch_types=[
            pltpu.VMEM((KNN, P), jnp.int32),
            pltpu.VMEM((KNN, P, C), jnp.float32),
            pltpu.SemaphoreType.DMA,
        ],
    )
    def sc_kernel(hf_hbm, idx_hbm, nbr_hbm, idx_v, rows_v, sem):
        wid = lax.axis_index("s") * 2 + lax.axis_index("c")
        fbase = wid * PW
        b = fbase // N
        n0 = fbase - b * N
        kbase = b * (KNN * N) + n0

        def sub_body(sub, _):
            pbase = fbase + sub * P
            for k in range(KNN):
                pltpu.sync_copy(idx_hbm.at[pl.ds(kbase + k * N + sub * P, P)],
                                idx_v.at[k])
            copies = [
                pltpu.async_copy(hf_hbm.at[idx_v.at[k]], rows_v.at[k], sem)
                for k in range(KNN)
            ]
            for cp in copies:
                cp.wait()
            for k in range(KNN):
                pltpu.sync_copy(rows_v.at[k], nbr_hbm.at[k, pl.ds(pbase, P)])
            return _

        lax.fori_loop(0, SUB, sub_body, None)

    return sc_kernel(hf, idxf)


def _edge_combine(nbr, ht, v2t, WaT, rb=512):
    B, N, C = ht.shape
    O = WaT.shape[1]
    nb = N // rb

    def body(nbr_ref, ctr_ref, v2_ref, wa_ref, mpv_ref, s1_ref, s2_ref):
        first = (pl.program_id(0) == 0) & (pl.program_id(1) == 0)

        @pl.when(first)
        def _():
            s1_ref[...] = jnp.zeros_like(s1_ref)
            s2_ref[...] = jnp.zeros_like(s2_ref)

        ctr = ctr_ref[0]
        v2 = v2_ref[0]
        wa = wa_ref[...]
        m = None
        p1 = 0.0
        p2 = 0.0
        for k in range(KNN):
            e = jnp.dot(nbr_ref[k] - ctr, wa,
                        preferred_element_type=jnp.float32)
            y = e + v2
            m = y if m is None else jnp.maximum(m, y)
            p1 = p1 + jnp.sum(y, axis=0, keepdims=True)
            p2 = p2 + jnp.sum(y * y, axis=0, keepdims=True)
        mpv_ref[0] = m
        s1_ref[...] += p1
        s2_ref[...] += p2

    return pl.pallas_call(
        body,
        grid=(B, nb),
        in_specs=[
            pl.BlockSpec((KNN, rb, C), lambda b, i, nb=nb: (0, b * nb + i, 0)),
            pl.BlockSpec((1, rb, C), lambda b, i: (b, i, 0)),
            pl.BlockSpec((1, rb, O), lambda b, i: (b, i, 0)),
            pl.BlockSpec((C, O), lambda b, i: (0, 0)),
        ],
        out_specs=[
            pl.BlockSpec((1, rb, O), lambda b, i: (b, i, 0)),
            pl.BlockSpec((1, O), lambda b, i: (0, 0)),
            pl.BlockSpec((1, O), lambda b, i: (0, 0)),
        ],
        out_shape=[
            jax.ShapeDtypeStruct((B, N, O), jnp.float32),
            jax.ShapeDtypeStruct((1, O), jnp.float32),
            jax.ShapeDtypeStruct((1, O), jnp.float32),
        ],
        interpret=_INTERPRET,
    )(nbr, ht, v2t, WaT)


def _normalize(mpv, s1, s2, g, bb):
    M, O = mpv.shape
    cnt = float(M * KNN)

    def body(mpv_ref, s1_ref, s2_ref, g_ref, b_ref, out_ref):
        mu = s1_ref[...] / cnt
        var = s2_ref[...] / cnt - mu * mu
        scale = g_ref[...] * lax.rsqrt(var + 1e-5)
        shift = b_ref[...] - mu * scale
        h = mpv_ref[...] * scale + shift
        out_ref[...] = jnp.where(h >= 0, h, 0.2 * h)

    return pl.pallas_call(
        body,
        out_shape=jax.ShapeDtypeStruct((M, O), jnp.float32),
        interpret=_INTERPRET,
    )(mpv, s1, s2, g, bb)


def _final_stats(h3, WcT, bc):
    B, N, C = h3.shape
    Oc = WcT.shape[1]

    def body(h_ref, w_ref, bc_ref, mx_ref, sm_ref, sq_ref):
        for b in range(B):
            z = jnp.dot(h_ref[b], w_ref[...],
                        preferred_element_type=jnp.float32) + bc_ref[...]
            a = jnp.where(z >= 0, z, 0.2 * z)
            mx_ref[b, :] = jnp.max(a, axis=0)
            sm_ref[b, :] = jnp.sum(a, axis=0)
            sq_ref[b, :] = jnp.sum(a * a, axis=0)

    return pl.pallas_call(
        body,
        out_shape=[
            jax.ShapeDtypeStruct((B, Oc), jnp.float32),
            jax.ShapeDtypeStruct((B, Oc), jnp.float32),
            jax.ShapeDtypeStruct((B, Oc), jnp.float32),
        ],
        interpret=_INTERPRET,
    )(h3, WcT, bc)


def _edge_conv_layer(ht, W, g, bb):
    B, N, C = ht.shape
    O = W.shape[0]
    Wa = W[:, :C]
    Wb = W[:, C:]
    idx, v2t = _knn_project(ht, Wb.T)
    nbr = _gather_nbr(ht.reshape(B * N, C), idx.reshape(-1))
    mpv, s1, s2 = _edge_combine(nbr, ht, v2t, Wa.T)
    h = _normalize(mpv.reshape(B * N, O), s1, s2, g.reshape(1, O),
                   bb.reshape(1, O))
    return h.reshape(B, N, O)


def kernel(x, W1, g1, b1, W2, g2, b2, W3, g3, b3, Wc, bc, gn, bnb):
    B, C0, N = x.shape
    CP = 128
    ht = jnp.transpose(x, (0, 2, 1))
    ht = jnp.pad(ht, ((0, 0), (0, 0), (0, CP - C0)))
    W1p = jnp.concatenate(
        [jnp.pad(W1[:, :C0], ((0, 0), (0, CP - C0))),
         jnp.pad(W1[:, C0:], ((0, 0), (0, CP - C0)))], axis=1)

    h = _edge_conv_layer(ht, W1p, g1, b1)
    h = _edge_conv_layer(h, W2, g2, b2)
    h = _edge_conv_layer(h, W3, g3, b3)

    Oc = Wc.shape[0]
    mx, sm, sq = _final_stats(h, Wc.T, bc.reshape(1, Oc))
    cnt = B * N
    mu = jnp.sum(sm, axis=0) / cnt
    var = jnp.sum(sq, axis=0) / cnt - mu * mu
    scale = gn / jnp.sqrt(var + 1e-5)
    shift = bnb - mu * scale
    m1 = mx * scale + shift
    m2 = (sm / N) * scale + shift
    return jnp.concatenate([m1, m2], axis=1)[:, :, None]


# speedup vs baseline: 29.4238x; 1.0031x over previous
import functools

import jax
import jax.numpy as jnp
from jax import lax
from jax.experimental import pallas as pl
from jax.experimental.pallas import tpu as pltpu
from jax.experimental.pallas import tpu_sc as plsc

KNN = 5
_INTERPRET = False


def _knn_project(ht, WbT, rb=512):
    B, N, C = ht.shape
    O = WbT.shape[1]
    nb = N // rb

    def body(hrow_ref, hfull_ref, wb_ref, idx_ref, v2_ref):
        b = pl.program_id(0)
        r = hrow_ref[0]
        f = hfull_ref[0]
        g = lax.dot_general(r, f, (((1,), (1,)), ((), ())),
                            preferred_element_type=jnp.float32)
        xx_r = jnp.sum(r * r, axis=1, keepdims=True)
        xx_f = jnp.sum(f * f, axis=1).reshape(1, N)
        d = 2.0 * g - xx_r - xx_f
        iota = lax.broadcasted_iota(jnp.int32, (rb, N), 1)
        off = b * N
        for t in range(KNN):
            amax = jnp.argmax(d, axis=1).astype(jnp.int32)
            idx_ref[0, :, t] = amax + off
            if t + 1 < KNN:
                d = jnp.where(iota == amax[:, None], -jnp.inf, d)
        v2_ref[0] = jnp.dot(r, wb_ref[...], preferred_element_type=jnp.float32)

    return pl.pallas_call(
        body,
        grid=(B, nb),
        in_specs=[
            pl.BlockSpec((1, rb, C), lambda b, i: (b, i, 0)),
            pl.BlockSpec((1, N, C), lambda b, i: (b, 0, 0)),
            pl.BlockSpec((C, O), lambda b, i: (0, 0)),
        ],
        out_specs=[
            pl.BlockSpec((1, rb, KNN), lambda b, i: (b, i, 0)),
            pl.BlockSpec((1, rb, O), lambda b, i: (b, i, 0)),
        ],
        out_shape=[
            jax.ShapeDtypeStruct((B, N, KNN), jnp.int32),
            jax.ShapeDtypeStruct((B, N, O), jnp.float32),
        ],
        interpret=_INTERPRET,
    )(ht, ht, WbT)


def _gather_nbr(hf, idxf):
    M, C = hf.shape
    NW = 32
    RT = M * KNN
    RW = RT // NW
    R = 128
    NCH = RW // R

    mesh = plsc.VectorSubcoreMesh(core_axis_name="c", subcore_axis_name="s")

    @functools.partial(
        pl.kernel,
        mesh=mesh,
        out_type=jax.ShapeDtypeStruct((RT, C), jnp.float32),
        scratch_types=[
            pltpu.VMEM((2, R), jnp.int32),
            pltpu.VMEM((2, R, C), jnp.float32),
            pltpu.SemaphoreType.DMA,
            pltpu.SemaphoreType.DMA,
            pltpu.SemaphoreType.DMA,
            pltpu.SemaphoreType.DMA,
        ],
    )
    def sc_kernel(hf_hbm, idx_hbm, nbr_hbm, idx_v, rows_v,
                  gsem0, gsem1, wsem0, wsem1):
        gsems = [gsem0, gsem1]
        wsems = [wsem0, wsem1]
        wid = lax.axis_index("s") * 2 + lax.axis_index("c")
        fbase = wid * RW

        def fire(j, slot):
            pltpu.sync_copy(idx_hbm.at[pl.ds(fbase + j * R, R)],
                            idx_v.at[slot])
            return pltpu.async_copy(hf_hbm.at[idx_v.at[slot]],
                                    rows_v.at[slot], gsems[slot])

        gcur = fire(0, 0)
        wbuf = [None, None]
        for j in range(NCH):
            s = j % 2
            gnext = None
            if j + 1 < NCH:
                os = 1 - s
                if wbuf[os] is not None:
                    wbuf[os].wait()
                gnext = fire(j + 1, os)
            gcur.wait()
            wbuf[s] = pltpu.async_copy(
                rows_v.at[s], nbr_hbm.at[pl.ds(fbase + j * R, R)], wsems[s])
            gcur = gnext
        for s in range(2):
            if wbuf[s] is not None:
                wbuf[s].wait()

    return sc_kernel(hf, idxf)


def _edge_combine(nbr, ht, v2t, WaT, rb=512):
    B, N, C = ht.shape
    O = WaT.shape[1]
    nb = N // rb
    nbr4 = nbr.reshape(B * N, KNN, C)

    def body(nbr_ref, ctr_ref, v2_ref, wa_ref, mpv_ref, s1_ref, s2_ref):
        first = (pl.program_id(0) == 0) & (pl.program_id(1) == 0)

        @pl.when(first)
        def _():
            s1_ref[...] = jnp.zeros_like(s1_ref)
            s2_ref[...] = jnp.zeros_like(s2_ref)

        ctr = ctr_ref[0]
        v2 = v2_ref[0]
        wa = wa_ref[...]
        m = None
        p1 = 0.0
        p2 = 0.0
        for k in range(KNN):
            e = jnp.dot(nbr_ref[:, k, :] - ctr, wa,
                        preferred_element_type=jnp.float32)
            y = e + v2
            m = y if m is None else jnp.maximum(m, y)
            p1 = p1 + jnp.sum(y, axis=0, keepdims=True)
            p2 = p2 + jnp.sum(y * y, axis=0, keepdims=True)
        mpv_ref[0] = m
        s1_ref[...] += p1
        s2_ref[...] += p2

    return pl.pallas_call(
        body,
        grid=(B, nb),
        in_specs=[
            pl.BlockSpec((rb, KNN, C), lambda b, i, nb=nb: (b * nb + i, 0, 0)),
            pl.BlockSpec((1, rb, C), lambda b, i: (b, i, 0)),
            pl.BlockSpec((1, rb, O), lambda b, i: (b, i, 0)),
            pl.BlockSpec((C, O), lambda b, i: (0, 0)),
        ],
        out_specs=[
            pl.BlockSpec((1, rb, O), lambda b, i: (b, i, 0)),
            pl.BlockSpec((1, O), lambda b, i: (0, 0)),
            pl.BlockSpec((1, O), lambda b, i: (0, 0)),
        ],
        out_shape=[
            jax.ShapeDtypeStruct((B, N, O), jnp.float32),
            jax.ShapeDtypeStruct((1, O), jnp.float32),
            jax.ShapeDtypeStruct((1, O), jnp.float32),
        ],
        interpret=_INTERPRET,
    )(nbr4, ht, v2t, WaT)


def _normalize(mpv, s1, s2, g, bb):
    M, O = mpv.shape
    cnt = float(M * KNN)

    def body(mpv_ref, s1_ref, s2_ref, g_ref, b_ref, out_ref):
        mu = s1_ref[...] / cnt
        var = s2_ref[...] / cnt - mu * mu
        scale = g_ref[...] * lax.rsqrt(var + 1e-5)
        shift = b_ref[...] - mu * scale
        h = mpv_ref[...] * scale + shift
        out_ref[...] = jnp.where(h >= 0, h, 0.2 * h)

    return pl.pallas_call(
        body,
        out_shape=jax.ShapeDtypeStruct((M, O), jnp.float32),
        interpret=_INTERPRET,
    )(mpv, s1, s2, g, bb)


def _final_stats(h3, WcT, bc):
    B, N, C = h3.shape
    Oc = WcT.shape[1]

    def body(h_ref, w_ref, bc_ref, mx_ref, sm_ref, sq_ref):
        for b in range(B):
            z = jnp.dot(h_ref[b], w_ref[...],
                        preferred_element_type=jnp.float32) + bc_ref[...]
            a = jnp.where(z >= 0, z, 0.2 * z)
            mx_ref[b, :] = jnp.max(a, axis=0)
            sm_ref[b, :] = jnp.sum(a, axis=0)
            sq_ref[b, :] = jnp.sum(a * a, axis=0)

    return pl.pallas_call(
        body,
        out_shape=[
            jax.ShapeDtypeStruct((B, Oc), jnp.float32),
            jax.ShapeDtypeStruct((B, Oc), jnp.float32),
            jax.ShapeDtypeStruct((B, Oc), jnp.float32),
        ],
        interpret=_INTERPRET,
    )(h3, WcT, bc)


def _edge_conv_layer(ht, W, g, bb):
    B, N, C = ht.shape
    O = W.shape[0]
    Wa = W[:, :C]
    Wb = W[:, C:]
    idx, v2t = _knn_project(ht, Wb.T)
    nbr = _gather_nbr(ht.reshape(B * N, C), idx.reshape(-1))
    mpv, s1, s2 = _edge_combine(nbr, ht, v2t, Wa.T)
    h = _normalize(mpv.reshape(B * N, O), s1, s2, g.reshape(1, O),
                   bb.reshape(1, O))
    return h.reshape(B, N, O)


def kernel(x, W1, g1, b1, W2, g2, b2, W3, g3, b3, Wc, bc, gn, bnb):
    B, C0, N = x.shape
    CP = 128
    ht = jnp.transpose(x, (0, 2, 1))
    ht = jnp.pad(ht, ((0, 0), (0, 0), (0, CP - C0)))
    W1p = jnp.concatenate(
        [jnp.pad(W1[:, :C0], ((0, 0), (0, CP - C0))),
         jnp.pad(W1[:, C0:], ((0, 0), (0, CP - C0)))], axis=1)

    h = _edge_conv_layer(ht, W1p, g1, b1)
    h = _edge_conv_layer(h, W2, g2, b2)
    h = _edge_conv_layer(h, W3, g3, b3)

    Oc = Wc.shape[0]
    mx, sm, sq = _final_stats(h, Wc.T, bc.reshape(1, Oc))
    cnt = B * N
    mu = jnp.sum(sm, axis=0) / cnt
    var = jnp.sum(sq, axis=0) / cnt - mu * mu
    scale = gn / jnp.sqrt(var + 1e-5)
    shift = bnb - mu * scale
    m1 = mx * scale + shift
    m2 = (sm / N) * scale + shift
    return jnp.concatenate([m1, m2], axis=1)[:, :, None]


# speedup vs baseline: 33.2422x; 1.1298x over previous
import functools

import jax
import jax.numpy as jnp
from jax import lax
from jax.experimental import pallas as pl
from jax.experimental.pallas import tpu as pltpu
from jax.experimental.pallas import tpu_sc as plsc

KNN = 5
_INTERPRET = False


def _knn_project(ht, rb=512):
    B, N, C = ht.shape
    nb = N // rb

    def body(hrow_ref, hfull_ref, idx_ref):
        b = pl.program_id(0)
        r = hrow_ref[0]
        f = hfull_ref[0]
        g = lax.dot_general(r, f, (((1,), (1,)), ((), ())),
                            preferred_element_type=jnp.float32)
        xx_r = jnp.sum(r * r, axis=1, keepdims=True)
        xx_f = jnp.sum(f * f, axis=1).reshape(1, N)
        d = 2.0 * g - xx_r - xx_f
        iota = lax.broadcasted_iota(jnp.int32, (rb, N), 1)
        off = b * N
        for t in range(KNN):
            amax = jnp.argmax(d, axis=1).astype(jnp.int32)
            idx_ref[0, t, :] = amax + off
            if t + 1 < KNN:
                d = jnp.where(iota == amax[:, None], -jnp.inf, d)

    return pl.pallas_call(
        body,
        grid=(B, nb),
        in_specs=[
            pl.BlockSpec((1, rb, C), lambda b, i: (b, i, 0)),
            pl.BlockSpec((1, N, C), lambda b, i: (b, 0, 0)),
        ],
        out_specs=[
            pl.BlockSpec((1, KNN, rb), lambda b, i: (b, 0, i)),
        ],
        out_shape=[
            jax.ShapeDtypeStruct((B, KNN, N), jnp.int32),
        ],
        interpret=_INTERPRET,
    )(ht, ht)[0]


def _gather_nbr(hf, idxf):
    M, C = hf.shape
    NW = 32
    RT = M * KNN
    RW = RT // NW
    R = 128
    NCH = RW // R

    mesh = plsc.VectorSubcoreMesh(core_axis_name="c", subcore_axis_name="s")

    @functools.partial(
        pl.kernel,
        mesh=mesh,
        out_type=jax.ShapeDtypeStruct((RT, C), jnp.float32),
        scratch_types=[
            pltpu.VMEM((2, R), jnp.int32),
            pltpu.VMEM((2, R, C), jnp.float32),
            pltpu.SemaphoreType.DMA,
            pltpu.SemaphoreType.DMA,
            pltpu.SemaphoreType.DMA,
            pltpu.SemaphoreType.DMA,
        ],
    )
    def sc_kernel(hf_hbm, idx_hbm, nbr_hbm, idx_v, rows_v,
                  gsem0, gsem1, wsem0, wsem1):
        gsems = [gsem0, gsem1]
        wsems = [wsem0, wsem1]
        wid = lax.axis_index("s") * 2 + lax.axis_index("c")
        fbase = wid * RW

        def fire(j, slot):
            pltpu.sync_copy(idx_hbm.at[pl.ds(fbase + j * R, R)],
                            idx_v.at[slot])
            return pltpu.async_copy(hf_hbm.at[idx_v.at[slot]],
                                    rows_v.at[slot], gsems[slot])

        gcur = fire(0, 0)
        wbuf = [None, None]
        for j in range(NCH):
            s = j % 2
            gnext = None
            if j + 1 < NCH:
                os = 1 - s
                if wbuf[os] is not None:
                    wbuf[os].wait()
                gnext = fire(j + 1, os)
            gcur.wait()
            wbuf[s] = pltpu.async_copy(
                rows_v.at[s], nbr_hbm.at[pl.ds(fbase + j * R, R)], wsems[s])
            gcur = gnext
        for s in range(2):
            if wbuf[s] is not None:
                wbuf[s].wait()

    return sc_kernel(hf, idxf)


def _edge_combine(nbr, ht, WT, rb=512):
    B, K, N, C = nbr.shape
    O = WT.shape[1]
    nb = N // rb

    def body(nbr_ref, ctr_ref, w_ref, mpv_ref, s1_ref, s2_ref):
        first = (pl.program_id(0) == 0) & (pl.program_id(1) == 0)

        @pl.when(first)
        def _():
            s1_ref[...] = jnp.zeros_like(s1_ref)
            s2_ref[...] = jnp.zeros_like(s2_ref)

        ctr = ctr_ref[0]
        ctr5 = jnp.concatenate([ctr] * KNN, axis=0)
        dmat = nbr_ref[0].reshape(KNN * rb, C) - ctr5
        feat = jnp.concatenate([dmat, ctr5], axis=1)
        y = jnp.dot(feat, w_ref[...],
                    preferred_element_type=jnp.float32)
        y3 = y.reshape(KNN, rb, O)
        m = y3[0]
        for k in range(1, KNN):
            m = jnp.maximum(m, y3[k])
        mpv_ref[0] = m
        s1_ref[...] += jnp.sum(y, axis=0, keepdims=True)
        s2_ref[...] += jnp.sum(y * y, axis=0, keepdims=True)

    return pl.pallas_call(
        body,
        grid=(B, nb),
        in_specs=[
            pl.BlockSpec((1, KNN, rb, C), lambda b, i: (b, 0, i, 0)),
            pl.BlockSpec((1, rb, C), lambda b, i: (b, i, 0)),
            pl.BlockSpec((2 * C, O), lambda b, i: (0, 0)),
        ],
        out_specs=[
            pl.BlockSpec((1, rb, O), lambda b, i: (b, i, 0)),
            pl.BlockSpec((1, O), lambda b, i: (0, 0)),
            pl.BlockSpec((1, O), lambda b, i: (0, 0)),
        ],
        out_shape=[
            jax.ShapeDtypeStruct((B, N, O), jnp.float32),
            jax.ShapeDtypeStruct((1, O), jnp.float32),
            jax.ShapeDtypeStruct((1, O), jnp.float32),
        ],
        interpret=_INTERPRET,
    )(nbr, ht, WT)


def _normalize(mpv, s1, s2, g, bb):
    M, O = mpv.shape
    cnt = float(M * KNN)

    def body(mpv_ref, s1_ref, s2_ref, g_ref, b_ref, out_ref):
        mu = s1_ref[...] / cnt
        var = s2_ref[...] / cnt - mu * mu
        scale = g_ref[...] * lax.rsqrt(var + 1e-5)
        shift = b_ref[...] - mu * scale
        h = mpv_ref[...] * scale + shift
        out_ref[...] = jnp.where(h >= 0, h, 0.2 * h)

    return pl.pallas_call(
        body,
        out_shape=jax.ShapeDtypeStruct((M, O), jnp.float32),
        interpret=_INTERPRET,
    )(mpv, s1, s2, g, bb)


def _final_stats(h3, WcT, bc):
    B, N, C = h3.shape
    Oc = WcT.shape[1]

    def body(h_ref, w_ref, bc_ref, mx_ref, sm_ref, sq_ref):
        for b in range(B):
            z = jnp.dot(h_ref[b], w_ref[...],
                        preferred_element_type=jnp.float32) + bc_ref[...]
            a = jnp.where(z >= 0, z, 0.2 * z)
            mx_ref[b, :] = jnp.max(a, axis=0)
            sm_ref[b, :] = jnp.sum(a, axis=0)
            sq_ref[b, :] = jnp.sum(a * a, axis=0)

    return pl.pallas_call(
        body,
        out_shape=[
            jax.ShapeDtypeStruct((B, Oc), jnp.float32),
            jax.ShapeDtypeStruct((B, Oc), jnp.float32),
            jax.ShapeDtypeStruct((B, Oc), jnp.float32),
        ],
        interpret=_INTERPRET,
    )(h3, WcT, bc)


def _edge_conv_layer(ht, W, g, bb):
    B, N, C = ht.shape
    O = W.shape[0]
    idx = _knn_project(ht)
    nbr = _gather_nbr(ht.reshape(B * N, C), idx.reshape(-1))
    mpv, s1, s2 = _edge_combine(nbr.reshape(B, KNN, N, C), ht, W.T)
    h = _normalize(mpv.reshape(B * N, O), s1, s2, g.reshape(1, O),
                   bb.reshape(1, O))
    return h.reshape(B, N, O)


def kernel(x, W1, g1, b1, W2, g2, b2, W3, g3, b3, Wc, bc, gn, bnb):
    B, C0, N = x.shape
    CP = 128
    ht = jnp.transpose(x, (0, 2, 1))
    ht = jnp.pad(ht, ((0, 0), (0, 0), (0, CP - C0)))
    W1p = jnp.concatenate(
        [jnp.pad(W1[:, :C0], ((0, 0), (0, CP - C0))),
         jnp.pad(W1[:, C0:], ((0, 0), (0, CP - C0)))], axis=1)

    h = _edge_conv_layer(ht, W1p, g1, b1)
    h = _edge_conv_layer(h, W2, g2, b2)
    h = _edge_conv_layer(h, W3, g3, b3)

    Oc = Wc.shape[0]
    mx, sm, sq = _final_stats(h, Wc.T, bc.reshape(1, Oc))
    cnt = B * N
    mu = jnp.sum(sm, axis=0) / cnt
    var = jnp.sum(sq, axis=0) / cnt - mu * mu
    scale = gn / jnp.sqrt(var + 1e-5)
    shift = bnb - mu * scale
    m1 = mx * scale + shift
    m2 = (sm / N) * scale + shift
    return jnp.concatenate([m1, m2], axis=1)[:, :, None]


# speedup vs baseline: 35.0633x; 1.0548x over previous
import functools

import jax
import jax.numpy as jnp
from jax import lax
from jax.experimental import pallas as pl
from jax.experimental.pallas import tpu as pltpu
from jax.experimental.pallas import tpu_sc as plsc

KNN = 5
_INTERPRET = False


def _knn_project(ht, rb=512):
    B, N, C = ht.shape
    nb = N // rb

    def body(hrow_ref, hfull_ref, xx_ref, idx_ref):
        b = pl.program_id(0)
        r = hrow_ref[0]
        f = hfull_ref[0]
        g = lax.dot_general(r, f, (((1,), (1,)), ((), ())),
                            preferred_element_type=jnp.float32)
        d = g - xx_ref[0]
        iota = lax.broadcasted_iota(jnp.int32, (rb, N), 1)
        off = b * N
        for t in range(KNN):
            amax = jnp.argmax(d, axis=1).astype(jnp.int32)
            idx_ref[0, t, :] = amax + off
            if t + 1 < KNN:
                d = jnp.where(iota == amax[:, None], -jnp.inf, d)

    xxh = 0.5 * jnp.sum(ht * ht, axis=2)

    return pl.pallas_call(
        body,
        grid=(B, nb),
        in_specs=[
            pl.BlockSpec((1, rb, C), lambda b, i: (b, i, 0)),
            pl.BlockSpec((1, N, C), lambda b, i: (b, 0, 0)),
            pl.BlockSpec((1, 1, N), lambda b, i: (b, 0, 0)),
        ],
        out_specs=[
            pl.BlockSpec((1, KNN, rb), lambda b, i: (b, 0, i)),
        ],
        out_shape=[
            jax.ShapeDtypeStruct((B, KNN, N), jnp.int32),
        ],
        interpret=_INTERPRET,
    )(ht, ht, xxh.reshape(B, 1, N))[0]


def _gather_nbr(hf, idxf):
    M, C = hf.shape
    NW = 32
    RT = M * KNN
    RW = RT // NW
    R = 128
    NCH = RW // R

    mesh = plsc.VectorSubcoreMesh(core_axis_name="c", subcore_axis_name="s")

    @functools.partial(
        pl.kernel,
        mesh=mesh,
        out_type=jax.ShapeDtypeStruct((RT, C), jnp.float32),
        scratch_types=[
            pltpu.VMEM((2, R), jnp.int32),
            pltpu.VMEM((2, R, C), jnp.float32),
            pltpu.SemaphoreType.DMA,
            pltpu.SemaphoreType.DMA,
            pltpu.SemaphoreType.DMA,
            pltpu.SemaphoreType.DMA,
        ],
    )
    def sc_kernel(hf_hbm, idx_hbm, nbr_hbm, idx_v, rows_v,
                  gsem0, gsem1, wsem0, wsem1):
        gsems = [gsem0, gsem1]
        wsems = [wsem0, wsem1]
        wid = lax.axis_index("s") * 2 + lax.axis_index("c")
        fbase = wid * RW

        def fire(j, slot):
            pltpu.sync_copy(idx_hbm.at[pl.ds(fbase + j * R, R)],
                            idx_v.at[slot])
            return pltpu.async_copy(hf_hbm.at[idx_v.at[slot]],
                                    rows_v.at[slot], gsems[slot])

        gcur = fire(0, 0)
        wbuf = [None, None]
        for j in range(NCH):
            s = j % 2
            gnext = None
            if j + 1 < NCH:
                os = 1 - s
                if wbuf[os] is not None:
                    wbuf[os].wait()
                gnext = fire(j + 1, os)
            gcur.wait()
            wbuf[s] = pltpu.async_copy(
                rows_v.at[s], nbr_hbm.at[pl.ds(fbase + j * R, R)], wsems[s])
            gcur = gnext
        for s in range(2):
            if wbuf[s] is not None:
                wbuf[s].wait()

    return sc_kernel(hf, idxf)


def _edge_combine(nbr, ht, WT, rb=512):
    B, K, N, C = nbr.shape
    O = WT.shape[1]
    nb = N // rb

    def body(nbr_ref, ctr_ref, w_ref, mpv_ref, s1_ref, s2_ref):
        first = (pl.program_id(0) == 0) & (pl.program_id(1) == 0)

        @pl.when(first)
        def _():
            s1_ref[...] = jnp.zeros_like(s1_ref)
            s2_ref[...] = jnp.zeros_like(s2_ref)

        ctr = ctr_ref[0]
        ctr5 = jnp.concatenate([ctr] * KNN, axis=0)
        dmat = nbr_ref[0].reshape(KNN * rb, C) - ctr5
        feat = jnp.concatenate([dmat, ctr5], axis=1)
        y = jnp.dot(feat, w_ref[...],
                    preferred_element_type=jnp.float32)
        y3 = y.reshape(KNN, rb, O)
        m = y3[0]
        for k in range(1, KNN):
            m = jnp.maximum(m, y3[k])
        mpv_ref[0] = m
        s1_ref[...] += jnp.sum(y, axis=0, keepdims=True)
        s2_ref[...] += jnp.sum(y * y, axis=0, keepdims=True)

    return pl.pallas_call(
        body,
        grid=(B, nb),
        in_specs=[
            pl.BlockSpec((1, KNN, rb, C), lambda b, i: (b, 0, i, 0)),
            pl.BlockSpec((1, rb, C), lambda b, i: (b, i, 0)),
            pl.BlockSpec((2 * C, O), lambda b, i: (0, 0)),
        ],
        out_specs=[
            pl.BlockSpec((1, rb, O), lambda b, i: (b, i, 0)),
            pl.BlockSpec((1, O), lambda b, i: (0, 0)),
            pl.BlockSpec((1, O), lambda b, i: (0, 0)),
        ],
        out_shape=[
            jax.ShapeDtypeStruct((B, N, O), jnp.float32),
            jax.ShapeDtypeStruct((1, O), jnp.float32),
            jax.ShapeDtypeStruct((1, O), jnp.float32),
        ],
        interpret=_INTERPRET,
    )(nbr, ht, WT)


def _normalize(mpv, s1, s2, g, bb):
    M, O = mpv.shape
    cnt = float(M * KNN)

    def body(mpv_ref, s1_ref, s2_ref, g_ref, b_ref, out_ref):
        mu = s1_ref[...] / cnt
        var = s2_ref[...] / cnt - mu * mu
        scale = g_ref[...] * lax.rsqrt(var + 1e-5)
        shift = b_ref[...] - mu * scale
        h = mpv_ref[...] * scale + shift
        out_ref[...] = jnp.where(h >= 0, h, 0.2 * h)

    return pl.pallas_call(
        body,
        out_shape=jax.ShapeDtypeStruct((M, O), jnp.float32),
        interpret=_INTERPRET,
    )(mpv, s1, s2, g, bb)


def _final_stats(h3, WcT, bc):
    B, N, C = h3.shape
    Oc = WcT.shape[1]

    def body(h_ref, w_ref, bc_ref, mx_ref, sm_ref, sq_ref):
        for b in range(B):
            z = jnp.dot(h_ref[b], w_ref[...],
                        preferred_element_type=jnp.float32) + bc_ref[...]
            a = jnp.where(z >= 0, z, 0.2 * z)
            mx_ref[b, :] = jnp.max(a, axis=0)
            sm_ref[b, :] = jnp.sum(a, axis=0)
            sq_ref[b, :] = jnp.sum(a * a, axis=0)

    return pl.pallas_call(
        body,
        out_shape=[
            jax.ShapeDtypeStruct((B, Oc), jnp.float32),
            jax.ShapeDtypeStruct((B, Oc), jnp.float32),
            jax.ShapeDtypeStruct((B, Oc), jnp.float32),
        ],
        interpret=_INTERPRET,
    )(h3, WcT, bc)


def _edge_conv_layer(ht, W, g, bb):
    B, N, C = ht.shape
    O = W.shape[0]
    idx = _knn_project(ht)
    nbr = _gather_nbr(ht.reshape(B * N, C), idx.reshape(-1))
    mpv, s1, s2 = _edge_combine(nbr.reshape(B, KNN, N, C), ht, W.T)
    h = _normalize(mpv.reshape(B * N, O), s1, s2, g.reshape(1, O),
                   bb.reshape(1, O))
    return h.reshape(B, N, O)


def kernel(x, W1, g1, b1, W2, g2, b2, W3, g3, b3, Wc, bc, gn, bnb):
    B, C0, N = x.shape
    CP = 128
    ht = jnp.transpose(x, (0, 2, 1))
    ht = jnp.pad(ht, ((0, 0), (0, 0), (0, CP - C0)))
    W1p = jnp.concatenate(
        [jnp.pad(W1[:, :C0], ((0, 0), (0, CP - C0))),
         jnp.pad(W1[:, C0:], ((0, 0), (0, CP - C0)))], axis=1)

    h = _edge_conv_layer(ht, W1p, g1, b1)
    h = _edge_conv_layer(h, W2, g2, b2)
    h = _edge_conv_layer(h, W3, g3, b3)

    Oc = Wc.shape[0]
    mx, sm, sq = _final_stats(h, Wc.T, bc.reshape(1, Oc))
    cnt = B * N
    mu = jnp.sum(sm, axis=0) / cnt
    var = jnp.sum(sq, axis=0) / cnt - mu * mu
    scale = gn / jnp.sqrt(var + 1e-5)
    shift = bnb - mu * scale
    m1 = mx * scale + shift
    m2 = (sm / N) * scale + shift
    return jnp.concatenate([m1, m2], axis=1)[:, :, None]
